# baseline (speedup 1.0000x reference)
import jax
import jax.numpy as jnp
from jax.experimental import pallas as pl
from jax.experimental.pallas import tpu as pltpu

F32 = jnp.float32
BF16 = jnp.bfloat16


def _dot(a, b, dims):
    return jax.lax.dot_general(a, b, (dims, ((), ())),
                               preferred_element_type=F32)


def _k1_body(x_ref, w_ref, b_ref, o_ref):
    wb = w_ref[...]
    ys = []
    for k in range(4):
        y = _dot(x_ref[k], wb, ((1,), (0,)))
        ys.append(jnp.maximum(y + b_ref[...], 0.0))
    o_ref[...] = jnp.maximum(jnp.maximum(ys[0], ys[1]),
                             jnp.maximum(ys[2], ys[3]))


def _conv1_pool(xcol4, W1r, b1):
    return pl.pallas_call(
        _k1_body,
        grid=(2,),
        in_specs=[
            pl.BlockSpec((4, 3136, 25), lambda i: (0, i, 0)),
            pl.BlockSpec((25, 96), lambda i: (0, 0)),
            pl.BlockSpec((1, 96), lambda i: (0, 0)),
        ],
        out_specs=pl.BlockSpec((3136, 96), lambda i: (i, 0)),
        out_shape=jax.ShapeDtypeStruct((6272, 96), F32),
        compiler_params=pltpu.CompilerParams(
            dimension_semantics=("parallel",)),
    )(xcol4, W1r, b1)


def _knn_body(x_ref, c_ref, o_ref):
    G, _, D = x_ref.shape
    ks = jax.lax.broadcasted_iota(jnp.int32, (32, 512), 1)
    for g in range(G):
        X = x_ref[g]
        n = jnp.sqrt(jnp.sum(X * X, axis=1, keepdims=True))
        xb = (X / jnp.maximum(n, 1e-12)).astype(BF16)
        Cb = c_ref[g].astype(BF16)
        s = _dot(xb, Cb, ((1,), (1,)))
        d = 1.0 - s
        dmin = jnp.min(d, axis=1, keepdims=True)
        idx = jnp.min(jnp.where(d == dmin, ks, 512), axis=1, keepdims=True)
        onehot = (ks == idx).astype(BF16)
        o_ref[g] = _dot(onehot, Cb, ((1,), (0,))).astype(BF16)


def _vq(xt, center):
    Cc, _, D = center.shape
    G = Cc // 12
    return pl.pallas_call(
        _knn_body,
        grid=(12,),
        in_specs=[
            pl.BlockSpec((G, 32, D), lambda i: (i, 0, 0)),
            pl.BlockSpec((G, 512, D), lambda i: (i, 0, 0)),
        ],
        out_specs=pl.BlockSpec((G, 32, D), lambda i: (i, 0, 0)),
        out_shape=jax.ShapeDtypeStruct((Cc, 32, D), BF16),
        compiler_params=pltpu.CompilerParams(
            dimension_semantics=("parallel",)),
    )(xt, center)


def _k3_body(x0_ref, x1_ref, w_ref, b_ref, o_ref):
    outs = []
    for xr in (x0_ref, x1_ref):
        acc = _dot(xr[0].reshape(3584, 480), w_ref[0], ((1,), (0,)))
        for dx in range(1, 5):
            acc = acc + _dot(xr[dx].reshape(3584, 480),
                             w_ref[dx], ((1,), (0,)))
        y = jnp.maximum(acc + b_ref[...], 0.0)
        y = y.reshape(32, 7, 2, 8, 192).max(axis=2)
        outs.append(y)
    o_ref[...] = jnp.maximum(outs[0], outs[1])


def _conv2_pool(xj0, xj1, W5, b2):
    return pl.pallas_call(
        _k3_body,
        in_specs=[
            pl.BlockSpec((5, 32, 14, 8, 480), lambda: (0, 0, 0, 0, 0)),
            pl.BlockSpec((5, 32, 14, 8, 480), lambda: (0, 0, 0, 0, 0)),
            pl.BlockSpec((5, 480, 192), lambda: (0, 0, 0)),
            pl.BlockSpec((1, 192), lambda: (0, 0)),
        ],
        out_specs=pl.BlockSpec((32, 7, 8, 192), lambda: (0, 0, 0, 0)),
        out_shape=jax.ShapeDtypeStruct((32, 7, 8, 192), F32),
    )(xj0, xj1, W5, b2)


def _k5_body(x_ref, w1_ref, b1_ref, w2_ref, b2_ref, o_ref, o1_ref):
    i = pl.program_id(0)
    wb = w1_ref[...].astype(BF16)
    o = _dot(x_ref[...], wb, ((1,), (1,)))
    o = jnp.maximum(o + b1_ref[:, pl.ds(i * 128, 128)], 0.0)
    o1_ref[:, pl.ds(i * 128, 128)] = o

    @pl.when(i == 7)
    def _():
        h = o1_ref[...].astype(BF16)
        w2 = w2_ref[...].astype(BF16)
        o_ref[...] = _dot(h, w2, ((1,), (1,))) + b2_ref[...]


def _fc(fcin, Wfc1, bfc1, Wfc2, bfc2):
    return pl.pallas_call(
        _k5_body,
        grid=(8,),
        in_specs=[
            pl.BlockSpec((32, 9408), lambda i: (0, 0)),
            pl.BlockSpec((128, 9408), lambda i: (i, 0)),
            pl.BlockSpec((1, 1024), lambda i: (0, 0)),
            pl.BlockSpec((1000, 1024), lambda i: (0, 0)),
            pl.BlockSpec((1, 1000), lambda i: (0, 0)),
        ],
        out_specs=pl.BlockSpec((32, 1000), lambda i: (0, 0)),
        out_shape=jax.ShapeDtypeStruct((32, 1000), F32),
        scratch_shapes=[pltpu.VMEM((32, 1024), F32)],
        compiler_params=pltpu.CompilerParams(
            dimension_semantics=("arbitrary",)),
    )(fcin, Wfc1, bfc1, Wfc2, bfc2)


def kernel(x, W1, b1, W2, b2, Wfc1, bfc1, Wfc2, bfc2, center0, center1):
    xp = jnp.pad(x[:, 0], ((0, 0), (2, 2), (2, 2)))
    xpar = xp.reshape(32, 16, 2, 16, 2).transpose(0, 2, 4, 1, 3)
    groups = []
    for i in range(2):
        for j in range(2):
            taps = []
            for dy in range(5):
                for dx in range(5):
                    ry, qy = (i + dy) % 2, (i + dy) // 2
                    rx, qx = (j + dx) % 2, (j + dx) // 2
                    taps.append(xpar[:, ry, rx, qy: qy + 14, qx: qx + 14])
            groups.append(jnp.stack(taps, axis=-1).reshape(6272, 25))
    xcol4 = jnp.stack(groups, axis=0).astype(BF16)
    W1r = W1.reshape(96, 25).T.astype(BF16)
    h1 = _conv1_pool(xcol4, W1r, b1.reshape(1, 96))

    h1t = h1.T.reshape(96, 32, 196)
    res1 = _vq(h1t, center0)

    h2in = jnp.transpose(res1, (1, 2, 0)).reshape(32, 14, 14, 96)
    hp2 = jnp.pad(h2in, ((0, 0), (2, 2), (2, 2), (0, 0)))
    xjs = []
    for j in range(2):
        per_dx = []
        for dx in range(5):
            sl = hp2[:, :, j + dx: j + dx + 13: 2, :]
            sl = jnp.pad(sl, ((0, 0), (0, 0), (0, 1), (0, 0)))
            cat = jnp.concatenate([sl[:, dy: dy + 14] for dy in range(5)],
                                  axis=3)
            per_dx.append(cat)
        xjs.append(jnp.stack(per_dx, axis=0))
    W5 = jnp.transpose(W2, (3, 2, 1, 0)).reshape(5, 480, 192).astype(BF16)
    h2 = _conv2_pool(xjs[0], xjs[1], W5, b2.reshape(1, 192))

    h2t = jnp.transpose(h2[:, :, :7, :].reshape(32, 49, 192), (2, 0, 1))
    res2 = _vq(h2t, center1)

    fcin = jnp.transpose(res2, (1, 0, 2)).reshape(32, 9408)
    return _fc(fcin, Wfc1, bfc1.reshape(1, 1024),
               Wfc2, bfc2.reshape(1, 1000))


# speedup vs baseline: 1.0008x; 1.0008x over previous
import jax
import jax.numpy as jnp
from jax.experimental import pallas as pl
from jax.experimental.pallas import tpu as pltpu

F32 = jnp.float32
BF16 = jnp.bfloat16


def _dot(a, b, dims):
    return jax.lax.dot_general(a, b, (dims, ((), ())),
                               preferred_element_type=F32)


def _k1_body(x_ref, w_ref, b_ref, o_ref):
    wb = w_ref[...]
    ys = []
    for k in range(4):
        y = _dot(x_ref[k], wb, ((1,), (0,)))
        ys.append(jnp.maximum(y + b_ref[...], 0.0))
    o_ref[...] = jnp.maximum(jnp.maximum(ys[0], ys[1]),
                             jnp.maximum(ys[2], ys[3]))


def _conv1_pool(xcol4, W1r, b1):
    return pl.pallas_call(
        _k1_body,
        grid=(2,),
        in_specs=[
            pl.BlockSpec((4, 3136, 25), lambda i: (0, i, 0)),
            pl.BlockSpec((25, 96), lambda i: (0, 0)),
            pl.BlockSpec((1, 96), lambda i: (0, 0)),
        ],
        out_specs=pl.BlockSpec((3136, 96), lambda i: (i, 0)),
        out_shape=jax.ShapeDtypeStruct((6272, 96), F32),
        compiler_params=pltpu.CompilerParams(
            dimension_semantics=("parallel",)),
    )(xcol4, W1r, b1)


def _knn_body(x_ref, c_ref, o_ref):
    G, _, D = x_ref.shape
    ks = jax.lax.broadcasted_iota(jnp.int32, (32, 512), 1)
    for g in range(G):
        X = x_ref[g]
        n = jnp.sqrt(jnp.sum(X * X, axis=1, keepdims=True))
        xb = (X / jnp.maximum(n, 1e-12)).astype(BF16)
        Cb = c_ref[g].astype(BF16)
        s = _dot(xb, Cb, ((1,), (1,)))
        d = 1.0 - s
        dc = [jax.lax.slice(d, (0, k * 128), (32, (k + 1) * 128))
              for k in range(4)]
        dmin = jnp.min(jnp.minimum(jnp.minimum(dc[0], dc[1]),
                                   jnp.minimum(dc[2], dc[3])),
                       axis=1, keepdims=True)
        iw = jnp.where(d == dmin, ks, 512)
        ic = [jax.lax.slice(iw, (0, k * 128), (32, (k + 1) * 128))
              for k in range(4)]
        idx = jnp.min(jnp.minimum(jnp.minimum(ic[0], ic[1]),
                                  jnp.minimum(ic[2], ic[3])),
                      axis=1, keepdims=True)
        onehot = (ks == idx).astype(BF16)
        o_ref[g] = _dot(onehot, Cb, ((1,), (0,))).astype(BF16)


def _vq(xt, center):
    Cc, _, D = center.shape
    G = Cc // 12
    return pl.pallas_call(
        _knn_body,
        grid=(12,),
        in_specs=[
            pl.BlockSpec((G, 32, D), lambda i: (i, 0, 0)),
            pl.BlockSpec((G, 512, D), lambda i: (i, 0, 0)),
        ],
        out_specs=pl.BlockSpec((G, 32, D), lambda i: (i, 0, 0)),
        out_shape=jax.ShapeDtypeStruct((Cc, 32, D), BF16),
        compiler_params=pltpu.CompilerParams(
            dimension_semantics=("parallel",)),
    )(xt, center)


def _k3_body(x0_ref, x1_ref, w_ref, b_ref, o_ref):
    outs = []
    for xr in (x0_ref, x1_ref):
        acc = _dot(xr[0].reshape(3584, 480), w_ref[0], ((1,), (0,)))
        for dx in range(1, 5):
            acc = acc + _dot(xr[dx].reshape(3584, 480),
                             w_ref[dx], ((1,), (0,)))
        y = jnp.maximum(acc + b_ref[...], 0.0)
        y = y.reshape(32, 7, 2, 8, 192).max(axis=2)
        outs.append(y)
    o_ref[...] = jnp.maximum(outs[0], outs[1])


def _conv2_pool(xj0, xj1, W5, b2):
    return pl.pallas_call(
        _k3_body,
        in_specs=[
            pl.BlockSpec((5, 32, 14, 8, 480), lambda: (0, 0, 0, 0, 0)),
            pl.BlockSpec((5, 32, 14, 8, 480), lambda: (0, 0, 0, 0, 0)),
            pl.BlockSpec((5, 480, 192), lambda: (0, 0, 0)),
            pl.BlockSpec((1, 192), lambda: (0, 0)),
        ],
        out_specs=pl.BlockSpec((32, 7, 8, 192), lambda: (0, 0, 0, 0)),
        out_shape=jax.ShapeDtypeStruct((32, 7, 8, 192), F32),
    )(xj0, xj1, W5, b2)


def _k5_body(x_ref, w1_ref, b1_ref, w2_ref, b2_ref, o_ref, o1_ref):
    i = pl.program_id(0)
    wb = w1_ref[...].astype(BF16)
    o = _dot(x_ref[...], wb, ((1,), (1,)))
    o = jnp.maximum(o + b1_ref[:, pl.ds(i * 128, 128)], 0.0)
    o1_ref[:, pl.ds(i * 128, 128)] = o

    @pl.when(i == 7)
    def _():
        h = o1_ref[...].astype(BF16)
        w2 = w2_ref[...].astype(BF16)
        o_ref[...] = _dot(h, w2, ((1,), (1,))) + b2_ref[...]


def _fc(fcin, Wfc1, bfc1, Wfc2, bfc2):
    return pl.pallas_call(
        _k5_body,
        grid=(8,),
        in_specs=[
            pl.BlockSpec((32, 9408), lambda i: (0, 0)),
            pl.BlockSpec((128, 9408), lambda i: (i, 0)),
            pl.BlockSpec((1, 1024), lambda i: (0, 0)),
            pl.BlockSpec((1000, 1024), lambda i: (0, 0)),
            pl.BlockSpec((1, 1000), lambda i: (0, 0)),
        ],
        out_specs=pl.BlockSpec((32, 1000), lambda i: (0, 0)),
        out_shape=jax.ShapeDtypeStruct((32, 1000), F32),
        scratch_shapes=[pltpu.VMEM((32, 1024), F32)],
        compiler_params=pltpu.CompilerParams(
            dimension_semantics=("arbitrary",)),
    )(fcin, Wfc1, bfc1, Wfc2, bfc2)


def kernel(x, W1, b1, W2, b2, Wfc1, bfc1, Wfc2, bfc2, center0, center1):
    xp = jnp.pad(x[:, 0], ((0, 0), (2, 2), (2, 2)))
    xpar = xp.reshape(32, 16, 2, 16, 2).transpose(0, 2, 4, 1, 3)
    groups = []
    for i in range(2):
        for j in range(2):
            taps = []
            for dy in range(5):
                for dx in range(5):
                    ry, qy = (i + dy) % 2, (i + dy) // 2
                    rx, qx = (j + dx) % 2, (j + dx) // 2
                    taps.append(xpar[:, ry, rx, qy: qy + 14, qx: qx + 14])
            groups.append(jnp.stack(taps, axis=-1).reshape(6272, 25))
    xcol4 = jnp.stack(groups, axis=0).astype(BF16)
    W1r = W1.reshape(96, 25).T.astype(BF16)
    h1 = _conv1_pool(xcol4, W1r, b1.reshape(1, 96))

    h1t = h1.T.reshape(96, 32, 196)
    res1 = _vq(h1t, center0)

    h2in = jnp.transpose(res1, (1, 2, 0)).reshape(32, 14, 14, 96)
    hp2 = jnp.pad(h2in, ((0, 0), (2, 2), (2, 2), (0, 0)))
    xjs = []
    for j in range(2):
        per_dx = []
        for dx in range(5):
            sl = hp2[:, :, j + dx: j + dx + 13: 2, :]
            sl = jnp.pad(sl, ((0, 0), (0, 0), (0, 1), (0, 0)))
            cat = jnp.concatenate([sl[:, dy: dy + 14] for dy in range(5)],
                                  axis=3)
            per_dx.append(cat)
        xjs.append(jnp.stack(per_dx, axis=0))
    W5 = jnp.transpose(W2, (3, 2, 1, 0)).reshape(5, 480, 192).astype(BF16)
    h2 = _conv2_pool(xjs[0], xjs[1], W5, b2.reshape(1, 192))

    h2t = jnp.transpose(h2[:, :, :7, :].reshape(32, 49, 192), (2, 0, 1))
    res2 = _vq(h2t, center1)

    fcin = jnp.transpose(res2, (1, 0, 2)).reshape(32, 9408)
    return _fc(fcin, Wfc1, bfc1.reshape(1, 1024),
               Wfc2, bfc2.reshape(1, 1000))


# speedup vs baseline: 1.1465x; 1.1456x over previous
import jax
import jax.numpy as jnp
from jax.experimental import pallas as pl
from jax.experimental.pallas import tpu as pltpu

F32 = jnp.float32
BF16 = jnp.bfloat16


def _dot(a, b, dims):
    return jax.lax.dot_general(a, b, (dims, ((), ())),
                               preferred_element_type=F32)


def _k1_body(x_ref, w_ref, b_ref, o_ref):
    wb = w_ref[...]
    ys = []
    for k in range(4):
        y = _dot(x_ref[k], wb, ((1,), (0,)))
        ys.append(jnp.maximum(y + b_ref[...], 0.0))
    o_ref[...] = jnp.maximum(jnp.maximum(ys[0], ys[1]),
                             jnp.maximum(ys[2], ys[3]))


def _conv1_pool(xcol4, W1r, b1):
    return pl.pallas_call(
        _k1_body,
        grid=(2,),
        in_specs=[
            pl.BlockSpec((4, 3136, 25), lambda i: (0, i, 0)),
            pl.BlockSpec((25, 96), lambda i: (0, 0)),
            pl.BlockSpec((1, 96), lambda i: (0, 0)),
        ],
        out_specs=pl.BlockSpec((3136, 96), lambda i: (i, 0)),
        out_shape=jax.ShapeDtypeStruct((6272, 96), F32),
        compiler_params=pltpu.CompilerParams(
            dimension_semantics=("parallel",)),
    )(xcol4, W1r, b1)


def _knn_body(x_ref, c_ref, o_ref):
    G, _, D = x_ref.shape
    ks = jax.lax.broadcasted_iota(jnp.int32, (32, 512), 1)
    for g in range(G):
        X = x_ref[g]
        n = jnp.sqrt(jnp.sum(X * X, axis=1, keepdims=True))
        xb = (X / jnp.maximum(n, 1e-12)).astype(BF16)
        Cb = c_ref[g].astype(BF16)
        s = _dot(xb, Cb, ((1,), (1,)))
        d = 1.0 - s
        dc = [jax.lax.slice(d, (0, k * 128), (32, (k + 1) * 128))
              for k in range(4)]
        dmin = jnp.min(jnp.minimum(jnp.minimum(dc[0], dc[1]),
                                   jnp.minimum(dc[2], dc[3])),
                       axis=1, keepdims=True)
        iw = jnp.where(d == dmin, ks, 512)
        ic = [jax.lax.slice(iw, (0, k * 128), (32, (k + 1) * 128))
              for k in range(4)]
        idx = jnp.min(jnp.minimum(jnp.minimum(ic[0], ic[1]),
                                  jnp.minimum(ic[2], ic[3])),
                      axis=1, keepdims=True)
        onehot = (ks == idx).astype(BF16)
        o_ref[g] = _dot(onehot, Cb, ((1,), (0,))).astype(BF16)


def _vq(xt, center):
    Cc, _, D = center.shape
    G = Cc // 12
    return pl.pallas_call(
        _knn_body,
        grid=(12,),
        in_specs=[
            pl.BlockSpec((G, 32, D), lambda i: (i, 0, 0)),
            pl.BlockSpec((G, 512, D), lambda i: (i, 0, 0)),
        ],
        out_specs=pl.BlockSpec((G, 32, D), lambda i: (i, 0, 0)),
        out_shape=jax.ShapeDtypeStruct((Cc, 32, D), BF16),
        compiler_params=pltpu.CompilerParams(
            dimension_semantics=("parallel",)),
    )(xt, center)


def _k3_body(pe_ref, po_ref, w_ref, b_ref, o_ref):
    planes = (pe_ref[...], po_ref[...])
    outs = []
    for j in range(2):
        acc = None
        for dx in range(5):
            pl_, q = (j + dx) % 2, (j + dx) // 2
            arr = planes[pl_][:, :, q: q + 8, :]
            cat = jnp.concatenate([arr[:, dy: dy + 14] for dy in range(5)],
                                  axis=3)
            d = _dot(cat.reshape(3584, 480), w_ref[dx], ((1,), (0,)))
            acc = d if acc is None else acc + d
        y = jnp.maximum(acc + b_ref[...], 0.0)
        y = y.reshape(32, 7, 2, 8, 192).max(axis=2)
        outs.append(y)
    o_ref[...] = jnp.maximum(outs[0], outs[1])


def _conv2_pool(pe, po, W5, b2):
    return pl.pallas_call(
        _k3_body,
        in_specs=[
            pl.BlockSpec((32, 18, 16, 96), lambda: (0, 0, 0, 0)),
            pl.BlockSpec((32, 18, 16, 96), lambda: (0, 0, 0, 0)),
            pl.BlockSpec((5, 480, 192), lambda: (0, 0, 0)),
            pl.BlockSpec((1, 192), lambda: (0, 0)),
        ],
        out_specs=pl.BlockSpec((32, 7, 8, 192), lambda: (0, 0, 0, 0)),
        out_shape=jax.ShapeDtypeStruct((32, 7, 8, 192), F32),
    )(pe, po, W5, b2)


def _k5_body(x_ref, w1_ref, b1_ref, w2_ref, b2_ref, o_ref, o1_ref):
    i = pl.program_id(0)
    wb = w1_ref[...].astype(BF16)
    o = _dot(x_ref[...], wb, ((1,), (1,)))
    o = jnp.maximum(o + b1_ref[:, pl.ds(i * 128, 128)], 0.0)
    o1_ref[:, pl.ds(i * 128, 128)] = o

    @pl.when(i == 7)
    def _():
        h = o1_ref[...].astype(BF16)
        w2 = w2_ref[...].astype(BF16)
        o_ref[...] = _dot(h, w2, ((1,), (1,))) + b2_ref[...]


def _fc(fcin, Wfc1, bfc1, Wfc2, bfc2):
    return pl.pallas_call(
        _k5_body,
        grid=(8,),
        in_specs=[
            pl.BlockSpec((32, 9408), lambda i: (0, 0)),
            pl.BlockSpec((128, 9408), lambda i: (i, 0)),
            pl.BlockSpec((1, 1024), lambda i: (0, 0)),
            pl.BlockSpec((1000, 1024), lambda i: (0, 0)),
            pl.BlockSpec((1, 1000), lambda i: (0, 0)),
        ],
        out_specs=pl.BlockSpec((32, 1000), lambda i: (0, 0)),
        out_shape=jax.ShapeDtypeStruct((32, 1000), F32),
        scratch_shapes=[pltpu.VMEM((32, 1024), F32)],
        compiler_params=pltpu.CompilerParams(
            dimension_semantics=("arbitrary",)),
    )(fcin, Wfc1, bfc1, Wfc2, bfc2)


def kernel(x, W1, b1, W2, b2, Wfc1, bfc1, Wfc2, bfc2, center0, center1):
    xp = jnp.pad(x[:, 0], ((0, 0), (2, 2), (2, 2)))
    xpar = xp.reshape(32, 16, 2, 16, 2).transpose(0, 2, 4, 1, 3)
    groups = []
    for i in range(2):
        for j in range(2):
            taps = []
            for dy in range(5):
                for dx in range(5):
                    ry, qy = (i + dy) % 2, (i + dy) // 2
                    rx, qx = (j + dx) % 2, (j + dx) // 2
                    taps.append(xpar[:, ry, rx, qy: qy + 14, qx: qx + 14])
            groups.append(jnp.stack(taps, axis=-1).reshape(6272, 25))
    xcol4 = jnp.stack(groups, axis=0).astype(BF16)
    W1r = W1.reshape(96, 25).T.astype(BF16)
    h1 = _conv1_pool(xcol4, W1r, b1.reshape(1, 96))

    h1t = h1.T.reshape(96, 32, 196)
    res1 = _vq(h1t, center0)

    h2in = jnp.transpose(res1, (1, 2, 0)).reshape(32, 14, 14, 96)
    hp2 = jnp.pad(h2in, ((0, 0), (2, 2), (2, 2), (0, 0)))
    pe = jnp.pad(hp2[:, :, 0::2, :], ((0, 0), (0, 0), (0, 7), (0, 0)))
    po = jnp.pad(hp2[:, :, 1::2, :], ((0, 0), (0, 0), (0, 7), (0, 0)))
    W5 = jnp.transpose(W2, (3, 2, 1, 0)).reshape(5, 480, 192).astype(BF16)
    h2 = _conv2_pool(pe, po, W5, b2.reshape(1, 192))

    h2t = jnp.transpose(h2[:, :, :7, :].reshape(32, 49, 192), (2, 0, 1))
    res2 = _vq(h2t, center1)

    fcin = jnp.transpose(res2, (1, 0, 2)).reshape(32, 9408)
    return _fc(fcin, Wfc1, bfc1.reshape(1, 1024),
               Wfc2, bfc2.reshape(1, 1000))


# speedup vs baseline: 1.1635x; 1.0148x over previous
import jax
import jax.numpy as jnp
from jax.experimental import pallas as pl
from jax.experimental.pallas import tpu as pltpu

F32 = jnp.float32
BF16 = jnp.bfloat16


def _dot(a, b, dims):
    return jax.lax.dot_general(a, b, (dims, ((), ())),
                               preferred_element_type=F32)


def _k1_body(x0_ref, x1_ref, w_ref, b_ref, o_ref):
    planes = (x0_ref[...], x1_ref[...])
    accs = [None, None]
    for i in range(2):
        for dy in range(5):
            pr, qy = (i + dy) % 2, (i + dy) // 2
            Xr = planes[pr][:, qy: qy + 14, :].reshape(448, 32).astype(BF16)
            d = _dot(Xr, w_ref[dy], ((1,), (0,)))
            accs[i] = d if accs[i] is None else accs[i] + d
    y = jnp.maximum(accs[0], accs[1]) + b_ref[...]
    y = jnp.maximum(y, 0.0)
    o_ref[...] = jnp.maximum(y[:, 0:1344], y[:, 1344:2688])


def _conv1_pool(xph0, xph1, B5, b1pat):
    return pl.pallas_call(
        _k1_body,
        in_specs=[
            pl.BlockSpec((32, 16, 32), lambda: (0, 0, 0)),
            pl.BlockSpec((32, 16, 32), lambda: (0, 0, 0)),
            pl.BlockSpec((5, 32, 2688), lambda: (0, 0, 0)),
            pl.BlockSpec((1, 2688), lambda: (0, 0)),
        ],
        out_specs=pl.BlockSpec((448, 1344), lambda: (0, 0)),
        out_shape=jax.ShapeDtypeStruct((448, 1344), F32),
    )(xph0, xph1, B5, b1pat)


def _knn_body(x_ref, c_ref, o_ref):
    G, _, D = x_ref.shape
    ks = jax.lax.broadcasted_iota(jnp.int32, (32, 512), 1)
    for g in range(G):
        X = x_ref[g]
        n = jnp.sqrt(jnp.sum(X * X, axis=1, keepdims=True))
        xb = (X / jnp.maximum(n, 1e-12)).astype(BF16)
        Cb = c_ref[g].astype(BF16)
        s = _dot(xb, Cb, ((1,), (1,)))
        d = 1.0 - s
        dc = [jax.lax.slice(d, (0, k * 128), (32, (k + 1) * 128))
              for k in range(4)]
        dmin = jnp.min(jnp.minimum(jnp.minimum(dc[0], dc[1]),
                                   jnp.minimum(dc[2], dc[3])),
                       axis=1, keepdims=True)
        iw = jnp.where(d == dmin, ks, 512)
        ic = [jax.lax.slice(iw, (0, k * 128), (32, (k + 1) * 128))
              for k in range(4)]
        idx = jnp.min(jnp.minimum(jnp.minimum(ic[0], ic[1]),
                                  jnp.minimum(ic[2], ic[3])),
                      axis=1, keepdims=True)
        onehot = (ks == idx).astype(BF16)
        o_ref[g] = _dot(onehot, Cb, ((1,), (0,))).astype(BF16)


def _vq(xt, center):
    Cc, _, D = center.shape
    G = Cc // 12
    return pl.pallas_call(
        _knn_body,
        grid=(12,),
        in_specs=[
            pl.BlockSpec((G, 32, D), lambda i: (i, 0, 0)),
            pl.BlockSpec((G, 512, D), lambda i: (i, 0, 0)),
        ],
        out_specs=pl.BlockSpec((G, 32, D), lambda i: (i, 0, 0)),
        out_shape=jax.ShapeDtypeStruct((Cc, 32, D), BF16),
        compiler_params=pltpu.CompilerParams(
            dimension_semantics=("parallel",)),
    )(xt, center)


def _k3_body(pe_ref, po_ref, w_ref, b_ref, o_ref):
    planes = (pe_ref[...], po_ref[...])
    outs = []
    for j in range(2):
        acc = None
        for dx in range(5):
            pl_, q = (j + dx) % 2, (j + dx) // 2
            arr = planes[pl_][:, :, q: q + 8, :]
            cat = jnp.concatenate([arr[:, dy: dy + 14] for dy in range(5)],
                                  axis=3)
            d = _dot(cat.reshape(3584, 480), w_ref[dx], ((1,), (0,)))
            acc = d if acc is None else acc + d
        y = jnp.maximum(acc + b_ref[...], 0.0)
        y = y.reshape(32, 7, 2, 8, 192).max(axis=2)
        outs.append(y)
    o_ref[...] = jnp.maximum(outs[0], outs[1])


def _conv2_pool(pe, po, W5, b2):
    return pl.pallas_call(
        _k3_body,
        in_specs=[
            pl.BlockSpec((32, 18, 16, 96), lambda: (0, 0, 0, 0)),
            pl.BlockSpec((32, 18, 16, 96), lambda: (0, 0, 0, 0)),
            pl.BlockSpec((5, 480, 192), lambda: (0, 0, 0)),
            pl.BlockSpec((1, 192), lambda: (0, 0)),
        ],
        out_specs=pl.BlockSpec((32, 7, 8, 192), lambda: (0, 0, 0, 0)),
        out_shape=jax.ShapeDtypeStruct((32, 7, 8, 192), F32),
    )(pe, po, W5, b2)


def _k5_body(x_ref, w1_ref, b1_ref, w2_ref, b2_ref, o_ref, o1_ref):
    i = pl.program_id(0)
    wb = w1_ref[...].astype(BF16)
    o = _dot(x_ref[...], wb, ((1,), (1,)))
    o = jnp.maximum(o + b1_ref[:, pl.ds(i * 128, 128)], 0.0)
    o1_ref[:, pl.ds(i * 128, 128)] = o

    @pl.when(i == 7)
    def _():
        h = o1_ref[...].astype(BF16)
        w2 = w2_ref[...].astype(BF16)
        o_ref[...] = _dot(h, w2, ((1,), (1,))) + b2_ref[...]


def _fc(fcin, Wfc1, bfc1, Wfc2, bfc2):
    return pl.pallas_call(
        _k5_body,
        grid=(8,),
        in_specs=[
            pl.BlockSpec((32, 9408), lambda i: (0, 0)),
            pl.BlockSpec((128, 9408), lambda i: (i, 0)),
            pl.BlockSpec((1, 1024), lambda i: (0, 0)),
            pl.BlockSpec((1000, 1024), lambda i: (0, 0)),
            pl.BlockSpec((1, 1000), lambda i: (0, 0)),
        ],
        out_specs=pl.BlockSpec((32, 1000), lambda i: (0, 0)),
        out_shape=jax.ShapeDtypeStruct((32, 1000), F32),
        scratch_shapes=[pltpu.VMEM((32, 1024), F32)],
        compiler_params=pltpu.CompilerParams(
            dimension_semantics=("arbitrary",)),
    )(fcin, Wfc1, bfc1, Wfc2, bfc2)


def kernel(x, W1, b1, W2, b2, Wfc1, bfc1, Wfc2, bfc2, center0, center1):
    xp = jnp.pad(x[:, 0], ((0, 0), (2, 2), (2, 2)))
    xph0, xph1 = xp[:, 0::2, :], xp[:, 1::2, :]
    win = jnp.arange(32)[:, None, None]
    jj = jnp.arange(2)[None, :, None]
    pw = jnp.arange(14)[None, None, :]
    dxm = win - jj - 2 * pw
    valid = (dxm >= 0) & (dxm < 5)
    Wt = jnp.transpose(W1[:, 0], (1, 2, 0))
    B5 = jnp.stack([jnp.where(valid[..., None],
                              Wt[dy][dxm.clip(0, 4)], 0.0)
                    for dy in range(5)])
    B5 = B5.reshape(5, 32, 2688).astype(BF16)
    b1pat = jnp.tile(b1, 28).reshape(1, 2688)
    h1 = _conv1_pool(xph0, xph1, B5, b1pat)

    h1t = jnp.transpose(h1.reshape(32, 14, 14, 96),
                        (3, 0, 1, 2)).reshape(96, 32, 196)
    res1 = _vq(h1t, center0)

    h2in = jnp.transpose(res1, (1, 2, 0)).reshape(32, 14, 14, 96)
    hp2 = jnp.pad(h2in, ((0, 0), (2, 2), (2, 2), (0, 0)))
    pe = jnp.pad(hp2[:, :, 0::2, :], ((0, 0), (0, 0), (0, 7), (0, 0)))
    po = jnp.pad(hp2[:, :, 1::2, :], ((0, 0), (0, 0), (0, 7), (0, 0)))
    W5 = jnp.transpose(W2, (3, 2, 1, 0)).reshape(5, 480, 192).astype(BF16)
    h2 = _conv2_pool(pe, po, W5, b2.reshape(1, 192))

    h2t = jnp.transpose(h2[:, :, :7, :].reshape(32, 49, 192), (2, 0, 1))
    res2 = _vq(h2t, center1)

    fcin = jnp.transpose(res2, (1, 0, 2)).reshape(32, 9408)
    return _fc(fcin, Wfc1, bfc1.reshape(1, 1024),
               Wfc2, bfc2.reshape(1, 1000))
